# Initial kernel scaffold; baseline (speedup 1.0000x reference)
import jax
import jax.numpy as jnp
from jax.experimental import pallas as pl


def kernel(x, weight, bias, gamma, beta):
    raise NotImplementedError("write your pallas kernel here")


# speedup vs baseline: 5.4302x; 5.4302x over previous
import functools

import jax
import jax.numpy as jnp
from jax import lax
from jax.experimental import pallas as pl
from jax.experimental.pallas import tpu as pltpu

_EPS = 1e-5
_NEG_SLOPE = 0.2
_KSZ = 3


def _fused_conv_gn_act_kernel(x_ref, w_ref, b_ref, g_ref, bt_ref, o_ref, *,
                              h, w, groups, eps, neg_slope):
    c = w_ref.shape[1]
    m = h * w

    acc = jnp.zeros((m, c), jnp.float32)
    for kh in range(_KSZ):
        for kw in range(_KSZ):
            xs = x_ref[0, kh:kh + h, kw:kw + w, :].reshape(m, c)
            acc = acc + jnp.dot(xs, w_ref[kh * _KSZ + kw],
                                preferred_element_type=jnp.float32)
    acc = acc + b_ref[0]

    cg = c // groups
    chan_g = lax.broadcasted_iota(jnp.int32, (c, groups), 0) // cg
    grp = lax.broadcasted_iota(jnp.int32, (c, groups), 1)
    ind = (chan_g == grp).astype(jnp.float32)

    s = jnp.sum(acc, axis=0, keepdims=True)
    q = jnp.sum(acc * acc, axis=0, keepdims=True)
    cnt = float(m * cg)
    mean_g = jnp.dot(s, ind, preferred_element_type=jnp.float32) / cnt
    ex2_g = jnp.dot(q, ind, preferred_element_type=jnp.float32) / cnt
    inv_g = lax.rsqrt(ex2_g - mean_g * mean_g + eps)

    mean = jnp.dot(mean_g, ind.T, preferred_element_type=jnp.float32)
    inv = jnp.dot(inv_g, ind.T, preferred_element_type=jnp.float32)

    gamma = g_ref[0].reshape(1, c)
    beta = bt_ref[0].reshape(1, c)
    scale = gamma * inv
    shift = beta - gamma * mean * inv

    z = acc * scale + shift
    z = jnp.where(z >= 0, z, neg_slope * z)
    o_ref[0] = z.reshape(h, w, c).astype(o_ref.dtype)


def kernel(x, weight, bias, gamma, beta):
    n, cin, h, w = x.shape
    cout = weight.shape[0]
    cin_g = weight.shape[1]
    groups = cin // cin_g
    cout_g = cout // groups

    xt = jnp.transpose(x, (0, 2, 3, 1))
    xp = jnp.pad(xt, ((0, 0), (1, 1), (1, 1), (0, 0))).astype(jnp.bfloat16)

    w5 = weight.reshape(groups, cout_g, cin_g, _KSZ, _KSZ)
    wbd = jnp.einsum('gh,goikl->klhigo', jnp.eye(groups, dtype=weight.dtype), w5)
    wt = wbd.reshape(_KSZ * _KSZ, cin, cout).astype(jnp.bfloat16)

    fused = functools.partial(_fused_conv_gn_act_kernel, h=h, w=w,
                              groups=groups, eps=_EPS, neg_slope=_NEG_SLOPE)

    out = pl.pallas_call(
        fused,
        out_shape=jax.ShapeDtypeStruct((n, h, w, cout), x.dtype),
        grid=(n,),
        in_specs=[
            pl.BlockSpec((1, h + 2, w + 2, cin), lambda i: (i, 0, 0, 0)),
            pl.BlockSpec((_KSZ * _KSZ, cin, cout), lambda i: (0, 0, 0)),
            pl.BlockSpec((1, cout), lambda i: (0, 0)),
            pl.BlockSpec((1, cout), lambda i: (0, 0)),
            pl.BlockSpec((1, cout), lambda i: (0, 0)),
        ],
        out_specs=pl.BlockSpec((1, h, w, cout), lambda i: (i, 0, 0, 0)),
        compiler_params=pltpu.CompilerParams(
            dimension_semantics=("parallel",),
            vmem_limit_bytes=64 * 1024 * 1024),
    )(xp, wt,
      bias.reshape(1, cout).astype(jnp.float32),
      gamma.reshape(1, cout).astype(jnp.float32),
      beta.reshape(1, cout).astype(jnp.float32))

    return jnp.transpose(out, (0, 3, 1, 2))
